# baseline (speedup 1.0000x reference)
import functools

import jax
import jax.numpy as jnp
from jax import lax
from jax.experimental import pallas as pl
from jax.experimental.pallas import tpu as pltpu
from jax.experimental.pallas import tpu_sc as plsc

_B, _G, _H, _NNZ, _NCOND, _Q = 4096, 1000, 1024, 131072, 2000, 256
_GP = 1024
_GH = _GP // 2
_NC, _NS, _L = 2, 16, 16
_RPC = _B // _NC
_XW = _RPC * _GH
_TW = _XW // _NS
_EPT = _NNZ // _NS
_CH = 128
_NCHUNK = _EPT // _CH
_PGT = _B // (_NC * _NS)
_ZW = 4096
_NZCOPY = _TW // _ZW


def _log1p_poly(v):
    z = v / (v + 2.0)
    u = z * z
    p = 1.0 / 11.0
    for c in (9.0, 7.0, 5.0, 3.0, 1.0):
        p = 1.0 / c + u * p
    return 2.0 * z * p


_SCH = 512
_NST = _EPT // _SCH


def _sc_body(vals_h, rows_h, cols_h, flag_h, pgid_h, ftab_h, ptab_h,
             x0_out, x1_out, pf_out, pg_out,
             val_st, row_st, col_st, sidx_big, sval_big, zeros_v, gidx_v,
             grow_v, xsh, sem_g, sem_lk, sem_s, sem_z):
    cid = lax.axis_index("c")
    sid = lax.axis_index("s")
    wid = sid * _NC + cid

    for k in range(_NST):
        src = pl.ds(sid * _SCH + k * (_NS * _SCH), _SCH)
        dst = pl.ds(k * _SCH, _SCH)
        pltpu.async_copy(vals_h.at[src], val_st.at[dst], sem_g)
        pltpu.async_copy(rows_h.at[src], row_st.at[dst], sem_g)
        pltpu.async_copy(cols_h.at[src], col_st.at[dst], sem_g)

    def zstore(i, c):
        zeros_v[pl.ds(i * _L, _L)] = jnp.zeros((_L,), jnp.float32)
        return c
    lax.fori_loop(0, _ZW // _L, zstore, 0)
    for k in range(_NZCOPY):
        pltpu.async_copy(zeros_v, xsh.at[pl.ds(sid * _TW + k * _ZW, _ZW)],
                         sem_z)

    for _ in range(3 * _NST):
        pltpu.make_async_copy(vals_h.at[pl.ds(0, _SCH)],
                              val_st.at[pl.ds(0, _SCH)], sem_g).wait()

    row_lo = cid * _RPC

    def wstore(i, c):
        val_st[pl.ds(i * _L, _L)] = _log1p_poly(val_st[pl.ds(i * _L, _L)])
        return c
    lax.fori_loop(0, _EPT // _L, wstore, 0)

    for p, x_out in ((0, x0_out), (1, x1_out)):
        col_lo = p * _GH
        for k in range(_NZCOPY):
            pltpu.make_async_copy(
                zeros_v, xsh.at[pl.ds(sid * _TW + k * _ZW, _ZW)],
                sem_z).wait()
        plsc.subcore_barrier()

        def chunk_body(j, nfired):
            rf = row_st[pl.ds(j * _CH, _L)][0]
            rl = row_st[pl.ds(j * _CH + _CH - _L, _L)][_L - 1]
            cond = (rl >= row_lo) & (rf < row_lo + _RPC)

            @pl.when(cond)
            def _fire_chunk():
                def vec_body(k, c2):
                    s = j * _CH + k * _L
                    w = val_st[pl.ds(s, _L)]
                    r = row_st[pl.ds(s, _L)]
                    cc = col_st[pl.ds(s, _L)] - col_lo
                    rr = r - row_lo
                    valid = ((rr >= 0) & (rr < _RPC)
                             & (cc >= 0) & (cc < _GH))
                    fidx = jnp.where(valid, rr * _GH + cc, _XW)
                    sidx_big[pl.ds(s, _L)] = fidx
                    sval_big[pl.ds(s, _L)] = jnp.where(valid, w, 0.0)
                    return c2
                lax.fori_loop(0, _CH // _L, vec_body, 0)
                pltpu.async_copy(
                    sval_big.at[pl.ds(j * _CH, _CH)],
                    xsh.at[sidx_big.at[pl.ds(j * _CH, _CH)]],
                    sem_s, add=True)
            return nfired + jnp.where(cond, 1, 0)

        nfired = lax.fori_loop(0, _NCHUNK, chunk_body, 0)

        if p == 0:
            base = wid * _PGT
            pltpu.sync_copy(flag_h.at[pl.ds(base, _PGT)], gidx_v)
            for q in range(2):
                pltpu.async_copy(ftab_h.at[gidx_v.at[pl.ds(q * 64, 64)]],
                                 grow_v, sem_lk).wait()
                pltpu.sync_copy(grow_v,
                                pf_out.at[pl.ds(base + q * 64, 64)])
            pltpu.sync_copy(pgid_h.at[pl.ds(base, _PGT)], gidx_v)
            for q in range(2):
                pltpu.async_copy(ptab_h.at[gidx_v.at[pl.ds(q * 64, 64)]],
                                 grow_v, sem_lk).wait()
                pltpu.sync_copy(grow_v,
                                pg_out.at[pl.ds(base + q * 64, 64)])

        def drain(j, c):
            pltpu.make_async_copy(sval_big.at[pl.ds(0, _CH)],
                                  xsh.at[sidx_big.at[pl.ds(0, _CH)]],
                                  sem_s).wait()
            return c
        lax.fori_loop(0, nfired, drain, 0)
        plsc.subcore_barrier()

        pltpu.sync_copy(xsh.at[pl.ds(sid * _TW, _TW)],
                        x_out.at[pl.ds(cid * _XW + sid * _TW, _TW)])
        if p == 0:
            for k in range(_NZCOPY):
                pltpu.async_copy(zeros_v,
                                 xsh.at[pl.ds(sid * _TW + k * _ZW, _ZW)],
                                 sem_z)


_sc_mesh = functools.partial(
    plsc.VectorSubcoreMesh, core_axis_name="c", subcore_axis_name="s")


@functools.lru_cache(maxsize=1)
def _build_sc_call():
    return functools.partial(
        pl.kernel,
        mesh=_sc_mesh(),
        out_type=(
            jax.ShapeDtypeStruct((_B * _GH,), jnp.float32),
            jax.ShapeDtypeStruct((_B * _GH,), jnp.float32),
            jax.ShapeDtypeStruct((_B, _Q), jnp.float32),
            jax.ShapeDtypeStruct((_B, _Q), jnp.float32),
        ),
        scratch_types=[
            pltpu.VMEM((_EPT,), jnp.float32),
            pltpu.VMEM((_EPT,), jnp.int32),
            pltpu.VMEM((_EPT,), jnp.int32),
            pltpu.VMEM((_EPT,), jnp.int32),
            pltpu.VMEM((_EPT,), jnp.float32),
            pltpu.VMEM((_ZW,), jnp.float32),
            pltpu.VMEM((_PGT,), jnp.int32),
            pltpu.VMEM((64, _Q), jnp.float32),
            pltpu.VMEM_SHARED((_XW + 2 * _L,), jnp.float32),
            pltpu.SemaphoreType.DMA,
            pltpu.SemaphoreType.DMA,
            pltpu.SemaphoreType.DMA,
            pltpu.SemaphoreType.DMA,
        ],
    )(_sc_body)


def _mm_body(e_ref, w_ref, o_ref):
    o_ref[...] = jnp.dot(e_ref[...], w_ref[...],
                         preferred_element_type=jnp.float32)


def _fuse_body(x0_ref, x1_ref, m0_ref, m1_ref, b1_ref, lng_ref, lnb_ref,
               wtop_ref, wmid_ref, wbot_ref, pf_ref, pg_ref, bf_ref, o_ref):
    acc = jnp.dot(x0_ref[...], m0_ref[...], preferred_element_type=jnp.float32)
    acc = acc + jnp.dot(x1_ref[...], m1_ref[...],
                        preferred_element_type=jnp.float32)
    acc = acc + b1_ref[...]
    h = jnp.maximum(acc, 0.0)
    mu = jnp.mean(h, axis=-1, keepdims=True)
    var = jnp.mean((h - mu) * (h - mu), axis=-1, keepdims=True)
    hn = (h - mu) * lax.rsqrt(var + 1e-5) * lng_ref[...] + lnb_ref[...]
    out = jnp.dot(hn, wtop_ref[...], preferred_element_type=jnp.float32)
    out = out + jnp.dot(pf_ref[...], wmid_ref[...],
                        preferred_element_type=jnp.float32)
    out = out + jnp.dot(pg_ref[...], wbot_ref[...],
                        preferred_element_type=jnp.float32)
    o_ref[...] = out + bf_ref[...]


_BLK = 256


def kernel(x_values, x_row_idx, x_col_idx, perturb_flag, perturb_gene_id,
           bb_gene_emb, W1, b1, ln_g, ln_b, flag_table, pert_table, Wf, bf):
    rows = x_row_idx.astype(jnp.int32)
    cols = x_col_idx.astype(jnp.int32)
    flag = perturb_flag.astype(jnp.int32)
    pgid = perturb_gene_id.astype(jnp.int32)

    x0_flat, x1_flat, pf, pg = _build_sc_call()(
        x_values, rows, cols, flag, pgid, flag_table, pert_table)
    X0 = x0_flat.reshape(_B, _GH)
    X1 = x1_flat.reshape(_B, _GH)

    e_pad = jnp.zeros((_GP, _H), jnp.float32).at[:_G].set(bb_gene_emb)
    M = pl.pallas_call(
        _mm_body,
        grid=(_GP // _BLK,),
        in_specs=[
            pl.BlockSpec((_BLK, _H), lambda i: (i, 0)),
            pl.BlockSpec((_H, _H), lambda i: (0, 0)),
        ],
        out_specs=pl.BlockSpec((_BLK, _H), lambda i: (i, 0)),
        out_shape=jax.ShapeDtypeStruct((_GP, _H), jnp.float32),
    )(e_pad, W1)

    out = pl.pallas_call(
        _fuse_body,
        grid=(_B // _BLK,),
        in_specs=[
            pl.BlockSpec((_BLK, _GH), lambda i: (i, 0)),
            pl.BlockSpec((_BLK, _GH), lambda i: (i, 0)),
            pl.BlockSpec((_GH, _H), lambda i: (0, 0)),
            pl.BlockSpec((_GH, _H), lambda i: (0, 0)),
            pl.BlockSpec((1, _H), lambda i: (0, 0)),
            pl.BlockSpec((1, _H), lambda i: (0, 0)),
            pl.BlockSpec((1, _H), lambda i: (0, 0)),
            pl.BlockSpec((_H, _H), lambda i: (0, 0)),
            pl.BlockSpec((_Q, _H), lambda i: (0, 0)),
            pl.BlockSpec((_Q, _H), lambda i: (0, 0)),
            pl.BlockSpec((_BLK, _Q), lambda i: (i, 0)),
            pl.BlockSpec((_BLK, _Q), lambda i: (i, 0)),
            pl.BlockSpec((1, _H), lambda i: (0, 0)),
        ],
        out_specs=pl.BlockSpec((_BLK, _H), lambda i: (i, 0)),
        out_shape=jax.ShapeDtypeStruct((_B, _H), jnp.float32),
    )(X0, X1, M[:_GH], M[_GH:], b1.reshape(1, _H), ln_g.reshape(1, _H),
      ln_b.reshape(1, _H), Wf[:_H], Wf[_H:_H + _Q], Wf[_H + _Q:],
      pf, pg, bf.reshape(1, _H))

    return out


# speedup vs baseline: 1.0542x; 1.0542x over previous
import functools

import jax
import jax.numpy as jnp
from jax import lax
from jax.experimental import pallas as pl
from jax.experimental.pallas import tpu as pltpu
from jax.experimental.pallas import tpu_sc as plsc

_B, _G, _H, _NNZ, _NCOND, _Q = 4096, 1000, 1024, 131072, 2000, 256
_GP = 1024
_GH = _GP // 2
_NC, _NS, _L = 2, 16, 16
_RPC = _B // _NC
_XW = _RPC * _GH
_TW = _XW // _NS
_EPT = _NNZ // _NS
_CH = 128
_NCHUNK = _EPT // _CH
_PGT = _B // (_NC * _NS)
_ZW = 4096
_NZCOPY = _TW // _ZW


def _log1p_poly(v):
    z = v / (v + 2.0)
    u = z * z
    p = 1.0 / 11.0
    for c in (9.0, 7.0, 5.0, 3.0, 1.0):
        p = 1.0 / c + u * p
    return 2.0 * z * p


_SCH = 512
_NST = _EPT // _SCH


def _sc_body(vals_h, rows_h, cols_h, flag_h, pgid_h, ftab_h, ptab_h,
             x0_out, x1_out, pf_out, pg_out,
             val_st, row_st, col_st, sidx_big, sval_big, zeros_v, gidx_v,
             grow_v, xsh, sem_g, sem_lk, sem_s, sem_z):
    cid = lax.axis_index("c")
    sid = lax.axis_index("s")
    wid = sid * _NC + cid

    for k in range(_NST):
        src = pl.ds(sid * _SCH + k * (_NS * _SCH), _SCH)
        dst = pl.ds(k * _SCH, _SCH)
        pltpu.async_copy(vals_h.at[src], val_st.at[dst], sem_g)
        pltpu.async_copy(rows_h.at[src], row_st.at[dst], sem_g)
        pltpu.async_copy(cols_h.at[src], col_st.at[dst], sem_g)

    def zstore(i, c):
        zeros_v[pl.ds(i * _L, _L)] = jnp.zeros((_L,), jnp.float32)
        return c
    lax.fori_loop(0, _ZW // _L, zstore, 0)
    for k in range(_NZCOPY):
        pltpu.async_copy(zeros_v, xsh.at[pl.ds(sid * _TW + k * _ZW, _ZW)],
                         sem_z)

    for _ in range(3 * _NST):
        pltpu.make_async_copy(vals_h.at[pl.ds(0, _SCH)],
                              val_st.at[pl.ds(0, _SCH)], sem_g).wait()

    row_lo = cid * _RPC

    for p, x_out in ((0, x0_out), (1, x1_out)):
        col_lo = p * _GH
        for k in range(_NZCOPY):
            pltpu.make_async_copy(
                zeros_v, xsh.at[pl.ds(sid * _TW + k * _ZW, _ZW)],
                sem_z).wait()
        plsc.subcore_barrier()

        def chunk_body(j, nfired):
            rf = row_st[pl.ds(j * _CH, _L)][0]
            rl = row_st[pl.ds(j * _CH + _CH - _L, _L)][_L - 1]
            cond = (rl >= row_lo) & (rf < row_lo + _RPC)

            @pl.when(cond)
            def _fire_chunk():
                def vec_body(k, c2):
                    s = j * _CH + k * _L
                    w = _log1p_poly(val_st[pl.ds(s, _L)])
                    r = row_st[pl.ds(s, _L)]
                    cc = col_st[pl.ds(s, _L)] - col_lo
                    rr = r - row_lo
                    valid = ((rr >= 0) & (rr < _RPC)
                             & (cc >= 0) & (cc < _GH))
                    fidx = jnp.where(valid, rr * _GH + cc, _XW)
                    sidx_big[pl.ds(s, _L)] = fidx
                    sval_big[pl.ds(s, _L)] = jnp.where(valid, w, 0.0)
                    return c2
                lax.fori_loop(0, _CH // _L, vec_body, 0)
                pltpu.async_copy(
                    sval_big.at[pl.ds(j * _CH, _CH)],
                    xsh.at[sidx_big.at[pl.ds(j * _CH, _CH)]],
                    sem_s, add=True)
            return nfired + jnp.where(cond, 1, 0)

        nfired = lax.fori_loop(0, _NCHUNK, chunk_body, 0)

        if p == 0:
            base = wid * _PGT
            pltpu.sync_copy(flag_h.at[pl.ds(base, _PGT)], gidx_v)
            for q in range(2):
                pltpu.async_copy(ftab_h.at[gidx_v.at[pl.ds(q * 64, 64)]],
                                 grow_v, sem_lk).wait()
                pltpu.sync_copy(grow_v,
                                pf_out.at[pl.ds(base + q * 64, 64)])
            pltpu.sync_copy(pgid_h.at[pl.ds(base, _PGT)], gidx_v)
            for q in range(2):
                pltpu.async_copy(ptab_h.at[gidx_v.at[pl.ds(q * 64, 64)]],
                                 grow_v, sem_lk).wait()
                pltpu.sync_copy(grow_v,
                                pg_out.at[pl.ds(base + q * 64, 64)])

        def drain(j, c):
            pltpu.make_async_copy(sval_big.at[pl.ds(0, _CH)],
                                  xsh.at[sidx_big.at[pl.ds(0, _CH)]],
                                  sem_s).wait()
            return c
        lax.fori_loop(0, nfired, drain, 0)
        plsc.subcore_barrier()

        pltpu.sync_copy(xsh.at[pl.ds(sid * _TW, _TW)],
                        x_out.at[pl.ds(cid * _XW + sid * _TW, _TW)])
        if p == 0:
            for k in range(_NZCOPY):
                pltpu.async_copy(zeros_v,
                                 xsh.at[pl.ds(sid * _TW + k * _ZW, _ZW)],
                                 sem_z)


_sc_mesh = functools.partial(
    plsc.VectorSubcoreMesh, core_axis_name="c", subcore_axis_name="s")


@functools.lru_cache(maxsize=1)
def _build_sc_call():
    return functools.partial(
        pl.kernel,
        mesh=_sc_mesh(),
        out_type=(
            jax.ShapeDtypeStruct((_B * _GH,), jnp.float32),
            jax.ShapeDtypeStruct((_B * _GH,), jnp.float32),
            jax.ShapeDtypeStruct((_B, _Q), jnp.float32),
            jax.ShapeDtypeStruct((_B, _Q), jnp.float32),
        ),
        scratch_types=[
            pltpu.VMEM((_EPT,), jnp.float32),
            pltpu.VMEM((_EPT,), jnp.int32),
            pltpu.VMEM((_EPT,), jnp.int32),
            pltpu.VMEM((_EPT,), jnp.int32),
            pltpu.VMEM((_EPT,), jnp.float32),
            pltpu.VMEM((_ZW,), jnp.float32),
            pltpu.VMEM((_PGT,), jnp.int32),
            pltpu.VMEM((64, _Q), jnp.float32),
            pltpu.VMEM_SHARED((_XW + 2 * _L,), jnp.float32),
            pltpu.SemaphoreType.DMA,
            pltpu.SemaphoreType.DMA,
            pltpu.SemaphoreType.DMA,
            pltpu.SemaphoreType.DMA,
        ],
    )(_sc_body)


def _mm_body(e_ref, w_ref, o_ref):
    o_ref[...] = jnp.dot(e_ref[...], w_ref[...],
                         preferred_element_type=jnp.float32)


def _fuse_body(x0_ref, x1_ref, m0_ref, m1_ref, b1_ref, lng_ref, lnb_ref,
               wtop_ref, wmid_ref, wbot_ref, pf_ref, pg_ref, bf_ref, o_ref):
    acc = jnp.dot(x0_ref[...], m0_ref[...], preferred_element_type=jnp.float32)
    acc = acc + jnp.dot(x1_ref[...], m1_ref[...],
                        preferred_element_type=jnp.float32)
    acc = acc + b1_ref[...]
    h = jnp.maximum(acc, 0.0)
    mu = jnp.mean(h, axis=-1, keepdims=True)
    var = jnp.mean((h - mu) * (h - mu), axis=-1, keepdims=True)
    hn = (h - mu) * lax.rsqrt(var + 1e-5) * lng_ref[...] + lnb_ref[...]
    out = jnp.dot(hn, wtop_ref[...], preferred_element_type=jnp.float32)
    out = out + jnp.dot(pf_ref[...], wmid_ref[...],
                        preferred_element_type=jnp.float32)
    out = out + jnp.dot(pg_ref[...], wbot_ref[...],
                        preferred_element_type=jnp.float32)
    o_ref[...] = out + bf_ref[...]


_BLK = 256


def kernel(x_values, x_row_idx, x_col_idx, perturb_flag, perturb_gene_id,
           bb_gene_emb, W1, b1, ln_g, ln_b, flag_table, pert_table, Wf, bf):
    rows = x_row_idx.astype(jnp.int32)
    cols = x_col_idx.astype(jnp.int32)
    flag = perturb_flag.astype(jnp.int32)
    pgid = perturb_gene_id.astype(jnp.int32)

    x0_flat, x1_flat, pf, pg = _build_sc_call()(
        x_values, rows, cols, flag, pgid, flag_table, pert_table)
    X0 = x0_flat.reshape(_B, _GH)
    X1 = x1_flat.reshape(_B, _GH)

    e_pad = jnp.zeros((_GP, _H), jnp.float32).at[:_G].set(bb_gene_emb)
    M = pl.pallas_call(
        _mm_body,
        grid=(_GP // _BLK,),
        in_specs=[
            pl.BlockSpec((_BLK, _H), lambda i: (i, 0)),
            pl.BlockSpec((_H, _H), lambda i: (0, 0)),
        ],
        out_specs=pl.BlockSpec((_BLK, _H), lambda i: (i, 0)),
        out_shape=jax.ShapeDtypeStruct((_GP, _H), jnp.float32),
    )(e_pad, W1)

    out = pl.pallas_call(
        _fuse_body,
        grid=(_B // _BLK,),
        in_specs=[
            pl.BlockSpec((_BLK, _GH), lambda i: (i, 0)),
            pl.BlockSpec((_BLK, _GH), lambda i: (i, 0)),
            pl.BlockSpec((_GH, _H), lambda i: (0, 0)),
            pl.BlockSpec((_GH, _H), lambda i: (0, 0)),
            pl.BlockSpec((1, _H), lambda i: (0, 0)),
            pl.BlockSpec((1, _H), lambda i: (0, 0)),
            pl.BlockSpec((1, _H), lambda i: (0, 0)),
            pl.BlockSpec((_H, _H), lambda i: (0, 0)),
            pl.BlockSpec((_Q, _H), lambda i: (0, 0)),
            pl.BlockSpec((_Q, _H), lambda i: (0, 0)),
            pl.BlockSpec((_BLK, _Q), lambda i: (i, 0)),
            pl.BlockSpec((_BLK, _Q), lambda i: (i, 0)),
            pl.BlockSpec((1, _H), lambda i: (0, 0)),
        ],
        out_specs=pl.BlockSpec((_BLK, _H), lambda i: (i, 0)),
        out_shape=jax.ShapeDtypeStruct((_B, _H), jnp.float32),
    )(X0, X1, M[:_GH], M[_GH:], b1.reshape(1, _H), ln_g.reshape(1, _H),
      ln_b.reshape(1, _H), Wf[:_H], Wf[_H:_H + _Q], Wf[_H + _Q:],
      pf, pg, bf.reshape(1, _H))

    return out
